# baseline (speedup 1.0000x reference)
import jax
import jax.numpy as jnp
from jax.experimental import pallas as pl


def _mm_body(x_ref, w_ref, o_ref):
    o_ref[...] = jnp.dot(x_ref[...], w_ref[...], preferred_element_type=jnp.float32)


def _mm(x, w, block_rows):
    m, k = x.shape
    _, n = w.shape
    assert m % block_rows == 0
    return pl.pallas_call(
        _mm_body,
        grid=(m // block_rows,),
        in_specs=[
            pl.BlockSpec((block_rows, k), lambda i: (i, 0)),
            pl.BlockSpec((k, n), lambda i: (0, 0)),
        ],
        out_specs=pl.BlockSpec((block_rows, n), lambda i: (i, 0)),
        out_shape=jax.ShapeDtypeStruct((m, n), jnp.float32),
    )(x, w)


def kernel(x, edge_index, edge_attr, params):
    n = x.shape[0]
    src, dst = edge_index[0], edge_index[1]
    loop = jnp.arange(n, dtype=src.dtype)
    row = jnp.concatenate([src, loop])
    col = jnp.concatenate([dst, loop])

    def conv(x, p, w):
        wful = jnp.concatenate([w, jnp.ones((n,), x.dtype)])
        deg = jax.ops.segment_sum(wful, col, num_segments=n)
        dinv = jax.lax.rsqrt(jnp.maximum(deg, 1e-30))
        norm = dinv[row] * wful * dinv[col]
        xw = _mm(x, p["lin"], 2000)
        out = jax.ops.segment_sum(norm[:, None] * xw[row], col, num_segments=n)
        return out + p["bias"]

    def gnorm(z, p, eps=1e-5):
        mean = jnp.mean(z, axis=0, keepdims=True)
        zc = z - mean * p["mean_scale"]
        var = jnp.mean(zc * zc, axis=0, keepdims=True)
        return p["weight"] * zc / jnp.sqrt(var + eps) + p["bias"]

    for i in range(9):
        p = params["convs"][i]
        h = jax.nn.relu(_mm(edge_attr, p["W1"], 8000) + p["b1"])
        w = jax.nn.sigmoid((h @ p["W2"] + p["b2"]).squeeze(-1))
        x = jax.nn.relu(conv(x, p, w))
        x = gnorm(x, params["norms"][i])

    ones_e = jnp.ones((src.shape[0],), dtype=x.dtype)
    mu = conv(x, params["conv_mu"], ones_e)
    logstd = conv(x, params["conv_logstd"], ones_e)
    return (mu, logstd)


# speedup vs baseline: 7.8536x; 7.8536x over previous
import functools

import jax
import jax.numpy as jnp
from jax import lax
from jax.experimental import pallas as pl
from jax.experimental.pallas import tpu as pltpu
from jax.experimental.pallas import tpu_sc as plsc

f32 = jnp.float32
i32 = jnp.int32

C = 128
PH = 128
BN = 2000
NLAYER = 9


def _emlp_body(ea_ref, w1_ref, b1_ref, w2_ref, b2_ref, w_ref, wt_ref, *, e, be):
    i = pl.program_id(0)
    h = jnp.maximum(
        jnp.dot(ea_ref[...], w1_ref[...], preferred_element_type=f32)
        + b1_ref[...], 0.0)
    logit = jnp.dot(h, w2_ref[...], preferred_element_type=f32) + b2_ref[...]
    s = 1.0 / (1.0 + jnp.exp(-logit))
    col = lax.broadcasted_iota(i32, (be, 16), 1)
    rowid = lax.broadcasted_iota(i32, (be, 16), 0) + i * be
    w = jnp.where(col < NLAYER, s, jnp.where(col == NLAYER, 1.0, 0.0))
    w = jnp.where(rowid < e, w, 0.0)
    w_ref[...] = w
    wt_ref[...] = w.T


def _edge_mlp(eap, w1cat, b1cat, w2bd, b2cat, e):
    epad = eap.shape[0]
    be = 2048
    grid = epad // be
    return pl.pallas_call(
        functools.partial(_emlp_body, e=e, be=be),
        grid=(grid,),
        in_specs=[
            pl.BlockSpec((be, 16), lambda i: (i, 0)),
            pl.BlockSpec((16, 288), lambda i: (0, 0)),
            pl.BlockSpec((1, 288), lambda i: (0, 0)),
            pl.BlockSpec((288, 16), lambda i: (0, 0)),
            pl.BlockSpec((1, 16), lambda i: (0, 0)),
        ],
        out_specs=[
            pl.BlockSpec((be, 16), lambda i: (i, 0)),
            pl.BlockSpec((16, be), lambda i: (0, i)),
        ],
        out_shape=[
            jax.ShapeDtypeStruct((epad, 16), f32),
            jax.ShapeDtypeStruct((16, epad), f32),
        ],
    )(eap, w1cat, b1cat, w2bd, b2cat)


def _dinv_body(d_ref, o_ref):
    o_ref[...] = lax.rsqrt(d_ref[0] + d_ref[1] + 1.0)


def _dinv(degp, n):
    return pl.pallas_call(
        _dinv_body,
        grid=(n // BN,),
        in_specs=[pl.BlockSpec((2, BN, 16), lambda i: (0, i, 0))],
        out_specs=pl.BlockSpec((BN, 16), lambda i: (i, 0)),
        out_shape=jax.ShapeDtypeStruct((n, 16), f32),
    )(degp)


def _mm_body(*refs, l, cout, colsplit, gnorm, ntot):
    if gnorm:
        h_ref, sums_ref, gw_ref, gb_ref, gms_ref, lin_ref, dinv_ref, o_ref = refs
        mean = sums_ref[0:1, :] / ntot
        ez2 = sums_ref[1:2, :] / ntot
        ms = gms_ref[...]
        var = ez2 - (2.0 * ms - ms * ms) * mean * mean
        a = gw_ref[...] * lax.rsqrt(var + 1e-5)
        b = gb_ref[...] - a * mean * ms
        u = refs[0][...] * a + b
    else:
        h_ref, lin_ref, dinv_ref, o_ref = refs
        u = h_ref[...]
    xw = jnp.dot(u, lin_ref[...], preferred_element_type=f32)
    y = xw * dinv_ref[...][:, l:l + 1]
    nb = y.shape[0]
    if colsplit:
        r = cout - PH
        o_ref[0] = y[:, :PH]
        o_ref[1, :, :r] = y[:, PH:]
        if r < PH:
            o_ref[1, :, r:] = jnp.zeros((nb, PH - r), f32)
    else:
        o_ref[:, :cout] = y
        if cout < PH:
            o_ref[:, cout:] = jnp.zeros((nb, PH - cout), f32)


def _mm(h, sums, gn, lin, dinv, l):
    n, cin = h.shape
    cout = lin.shape[1]
    colsplit = cout > PH
    gnorm = sums is not None
    body = functools.partial(_mm_body, l=l, cout=cout, colsplit=colsplit,
                             gnorm=gnorm, ntot=float(n))
    in_specs = [pl.BlockSpec((BN, cin), lambda i: (i, 0))]
    args = [h]
    if gnorm:
        in_specs += [
            pl.BlockSpec((2, cin), lambda i: (0, 0)),
            pl.BlockSpec((1, cin), lambda i: (0, 0)),
            pl.BlockSpec((1, cin), lambda i: (0, 0)),
            pl.BlockSpec((1, cin), lambda i: (0, 0)),
        ]
        args += [sums, gn["weight"][None], gn["bias"][None],
                 gn["mean_scale"][None]]
    in_specs += [
        pl.BlockSpec((cin, cout), lambda i: (0, 0)),
        pl.BlockSpec((BN, 16), lambda i: (i, 0)),
    ]
    args += [lin, dinv]
    if colsplit:
        out_specs = pl.BlockSpec((2, BN, PH), lambda i: (0, i, 0))
        out_shape = jax.ShapeDtypeStruct((2, n, PH), f32)
    else:
        out_specs = pl.BlockSpec((BN, PH), lambda i: (i, 0))
        out_shape = jax.ShapeDtypeStruct((n, PH), f32)
    return pl.pallas_call(
        body,
        grid=(n // BN,),
        in_specs=in_specs,
        out_specs=out_specs,
        out_shape=out_shape,
    )(*args)


def _post_body(agg_ref, y_ref, dinv_ref, bias_ref, h_ref, sums_ref, *,
               l, cout, colsplit, relu):
    i = pl.program_id(0)
    if colsplit:
        a0 = agg_ref[0] + y_ref[0]
        a1 = agg_ref[1] + y_ref[1]
        z = jnp.concatenate([a0, a1[:, :cout - PH]], axis=1)
    else:
        z = (agg_ref[0] + agg_ref[1] + y_ref[...])[:, :cout]
    z = z * dinv_ref[...][:, l:l + 1] + bias_ref[...]
    if relu:
        z = jnp.maximum(z, 0.0)
    h_ref[...] = z
    ps = jnp.concatenate([jnp.sum(z, axis=0, keepdims=True),
                          jnp.sum(z * z, axis=0, keepdims=True)], axis=0)

    @pl.when(i == 0)
    def _():
        sums_ref[...] = ps

    @pl.when(i > 0)
    def _():
        sums_ref[...] += ps


def _post(agg, y, dinv, bias, l, cout):
    colsplit = y.ndim == 3
    n = y.shape[1] if colsplit else y.shape[0]
    body = functools.partial(_post_body, l=l, cout=cout, colsplit=colsplit,
                             relu=True)
    y_spec = (pl.BlockSpec((2, BN, PH), lambda i: (0, i, 0)) if colsplit
              else pl.BlockSpec((BN, PH), lambda i: (i, 0)))
    return pl.pallas_call(
        body,
        grid=(n // BN,),
        in_specs=[
            pl.BlockSpec((2, BN, PH), lambda i: (0, i, 0)),
            y_spec,
            pl.BlockSpec((BN, 16), lambda i: (i, 0)),
            pl.BlockSpec((1, cout), lambda i: (0, 0)),
        ],
        out_specs=[
            pl.BlockSpec((BN, cout), lambda i: (i, 0)),
            pl.BlockSpec((2, cout), lambda i: (0, 0)),
        ],
        out_shape=[
            jax.ShapeDtypeStruct((n, cout), f32),
            jax.ShapeDtypeStruct((2, cout), f32),
        ],
    )(agg, y, dinv, bias)


def _final_body(agg_ref, y_ref, dinv_ref, bias_ref, mu_ref, ls_ref, *, l):
    z = agg_ref[0] + agg_ref[1] + y_ref[...]
    z = z * dinv_ref[...][:, l:l + 1] + bias_ref[...]
    mu_ref[...] = z[:, :64]
    ls_ref[...] = z[:, 64:]


def _final(agg, y, dinv, bias):
    n = y.shape[0]
    body = functools.partial(_final_body, l=NLAYER)
    return pl.pallas_call(
        body,
        grid=(n // BN,),
        in_specs=[
            pl.BlockSpec((2, BN, PH), lambda i: (0, i, 0)),
            pl.BlockSpec((BN, PH), lambda i: (i, 0)),
            pl.BlockSpec((BN, 16), lambda i: (i, 0)),
            pl.BlockSpec((1, 128), lambda i: (0, 0)),
        ],
        out_specs=[
            pl.BlockSpec((BN, 64), lambda i: (i, 0)),
            pl.BlockSpec((BN, 64), lambda i: (i, 0)),
        ],
        out_shape=[
            jax.ShapeDtypeStruct((n, 64), f32),
            jax.ShapeDtypeStruct((n, 64), f32),
        ],
    )(agg, y, dinv, bias)


_MESH = dict(core_axis_name="c", subcore_axis_name="s")


def _zero_rows(zbuf, nrows, nj):
    def zrow(i, carry):
        for j in range(nj):
            zbuf[i, pl.ds(j * 16, 16)] = jnp.zeros((16,), f32)
        return carry
    lax.fori_loop(0, nrows, zrow, 0, unroll=4)


def _init_acc(zbuf, acc, row0, rpt):
    nfull = rpt // 128
    for k in range(nfull):
        pltpu.sync_copy(zbuf, acc.at[pl.ds(row0 + k * 128, 128)])
    rem = rpt - nfull * 128
    if rem:
        pltpu.sync_copy(zbuf.at[pl.ds(0, rem)],
                        acc.at[pl.ds(row0 + nfull * 128, rem)])


def _build_deg(epad, npad):
    e_sc = epad // 2
    e_tile = e_sc // 16
    g = e_tile // C
    rpt = npad // 16
    mesh = plsc.VectorSubcoreMesh(**_MESH)

    @functools.partial(
        pl.kernel, mesh=mesh,
        out_type=jax.ShapeDtypeStruct((2, npad, 16), f32),
        scratch_types=[
            pltpu.VMEM((128, 16), f32),
            pltpu.VMEM((1, C), i32),
            pltpu.VMEM((C, 16), f32),
            pltpu.VMEM_SHARED((npad, 16), f32),
        ])
    def deg_kernel(wmat, dstp, out, zbuf, dstbuf, wbuf, acc):
        h = lax.axis_index("c")
        s = lax.axis_index("s")
        _zero_rows(zbuf, 128, 1)
        row0 = s * rpt
        _init_acc(zbuf, acc, row0, rpt)
        plsc.subcore_barrier()
        base = h * e_sc + s * e_tile

        def body(gg, carry):
            e0 = pl.multiple_of(base + gg * C, C)
            pltpu.sync_copy(dstp.at[pl.ds(e0, C)], dstbuf.at[0])
            pltpu.sync_copy(wmat.at[pl.ds(e0, C)], wbuf)
            pltpu.sync_copy(wbuf, acc.at[dstbuf.at[0]], add=True)
            return carry
        lax.fori_loop(0, g, body, 0)
        plsc.subcore_barrier()
        pltpu.sync_copy(acc.at[pl.ds(row0, rpt)], out.at[h, pl.ds(row0, rpt)])

    return deg_kernel


def _build_prop(epad, npad, colsplit, weighted):
    e_tile = (epad if colsplit else epad // 2) // 16
    g = e_tile // C
    rpt = npad // 16
    nj = PH // 16
    mesh = plsc.VectorSubcoreMesh(**_MESH)

    scratch = [
        pltpu.VMEM((128, PH), f32),
        pltpu.VMEM((C,), i32),
        pltpu.VMEM((1, C), i32),
        pltpu.VMEM((C,), f32),
        pltpu.VMEM((C, PH), f32),
        pltpu.VMEM_SHARED((npad, PH), f32),
    ]

    @functools.partial(
        pl.kernel, mesh=mesh,
        out_type=jax.ShapeDtypeStruct((2, npad, PH), f32),
        scratch_types=scratch)
    def prop_kernel(y2f, srcp, dstp, wrow, out, zbuf, idxbuf, dstbuf, wbuf,
                    rowbuf, acc):
        h = lax.axis_index("c")
        s = lax.axis_index("s")
        _zero_rows(zbuf, 128, nj)
        row0 = s * rpt
        _init_acc(zbuf, acc, row0, rpt)
        plsc.subcore_barrier()
        base = (s if colsplit else h * 16 + s) * e_tile

        def body(gg, carry):
            e0 = pl.multiple_of(base + gg * C, C)
            if colsplit:
                pltpu.sync_copy(srcp.at[h, pl.ds(e0, C)], idxbuf)
            else:
                pltpu.sync_copy(srcp.at[0, pl.ds(e0, C)], idxbuf)
            pltpu.sync_copy(dstp.at[pl.ds(e0, C)], dstbuf.at[0])
            if weighted:
                pltpu.sync_copy(wrow.at[pl.ds(e0, C)], wbuf)
            pltpu.sync_copy(y2f.at[idxbuf], rowbuf)
            if weighted:
                def scale(q, carry2):
                    wv = wbuf[pl.ds(q * 16, 16)]
                    for k in range(16):
                        w = wv[k]
                        e1 = q * 16 + k
                        for j in range(nj):
                            sl = pl.ds(j * 16, 16)
                            rowbuf[e1, sl] = rowbuf[e1, sl] * w
                    return carry2
                lax.fori_loop(0, C // 16, scale, 0)
            pltpu.sync_copy(rowbuf, acc.at[dstbuf.at[0]], add=True)
            return carry
        lax.fori_loop(0, g, body, 0)
        plsc.subcore_barrier()
        pltpu.sync_copy(acc.at[pl.ds(row0, rpt)], out.at[h, pl.ds(row0, rpt)])

    return prop_kernel


def kernel(x, edge_index, edge_attr, params):
    n, d_in = x.shape
    e = edge_index.shape[1]
    epad = ((e + 2047) // 2048) * 2048
    npad = ((n + 128) // 128) * 128

    src = edge_index[0]
    dst = edge_index[1]
    srcp = jnp.pad(src, (0, epad - e))
    dstp = jnp.pad(dst, (0, epad - e), constant_values=n)
    src2 = jnp.stack([srcp, srcp + n])
    src1 = srcp[None]
    eap = jnp.pad(edge_attr, ((0, epad - e), (0, 0)))

    convs = params["convs"]
    norms = params["norms"]
    w1cat = jnp.concatenate([c["W1"] for c in convs], axis=1)
    b1cat = jnp.concatenate([c["b1"] for c in convs])[None]
    w2s = jnp.stack([c["W2"][:, 0] for c in convs])
    eye = jnp.eye(16, dtype=f32)[:NLAYER]
    w2bd = (w2s[:, :, None] * eye[:, None, :]).reshape(288, 16)
    b2cat = jnp.concatenate(
        [jnp.stack([c["b2"][0] for c in convs]), jnp.zeros((7,), f32)])[None]

    wmat, wmat_t = _edge_mlp(eap, w1cat, b1cat, w2bd, b2cat, e)

    degp = _build_deg(epad, npad)(wmat, dstp)
    dinv = _dinv(degp, n)

    h = x
    sums = None
    for l in range(NLAYER):
        p = convs[l]
        cout = p["lin"].shape[1]
        colsplit = cout > PH
        gn = norms[l - 1] if l > 0 else None
        y2 = _mm(h, sums, gn, p["lin"], dinv, l)
        prop = _build_prop(epad, npad, colsplit, weighted=True)
        yflat = y2.reshape(2 * n, PH) if colsplit else y2
        agg = prop(yflat, src2 if colsplit else src1, dstp, wmat_t[l])
        h, sums = _post(agg, y2, dinv, p["bias"][None], l, cout)

    lincat = jnp.concatenate(
        [params["conv_mu"]["lin"], params["conv_logstd"]["lin"]], axis=1)
    bcat = jnp.concatenate(
        [params["conv_mu"]["bias"], params["conv_logstd"]["bias"]])[None]
    y2 = _mm(h, sums, norms[NLAYER - 1], lincat, dinv, NLAYER)
    prop = _build_prop(epad, npad, colsplit=False, weighted=False)
    agg = prop(y2, src1, dstp, wmat_t[NLAYER])
    mu, logstd = _final(agg, y2, dinv, bcat)
    return (mu, logstd)
